# baseline (speedup 1.0000x reference)
import functools

import jax
import jax.numpy as jnp
from jax import lax
from jax.experimental import pallas as pl
from jax.experimental.pallas import tpu as pltpu
from jax.experimental.pallas import tpu_sc as plsc

_NUM_CORES = 2
_NUM_SUBCORES = 16
_NW = _NUM_CORES * _NUM_SUBCORES
_L = 16

_VB = 256
_UNROLL = 8


def _make_sc_gather(total: int):
    assert total % _L == 0
    nv_total = total // _L
    base_nv = nv_total // _NW
    extra = nv_total % _NW
    max_nv = base_nv + (1 if extra else 0)
    nb = -(-max_nv // _VB)
    assert base_nv >= _VB

    mesh = plsc.VectorSubcoreMesh(
        core_axis_name="c", subcore_axis_name="s", num_cores=_NUM_CORES
    )

    @functools.partial(
        pl.kernel,
        out_type=jax.ShapeDtypeStruct((total,), jnp.float32),
        mesh=mesh,
        scratch_types=[
            pltpu.VMEM((_L,), jnp.float32),
            pltpu.VMEM((_VB * _L,), jnp.int32),
            pltpu.VMEM((_VB * _L,), jnp.float32),
        ],
        compiler_params=pltpu.CompilerParams(needs_layout_passes=False),
    )
    def sc_gather(w_hbm, idx_hbm, out_hbm, wv, idxbuf, outbuf):
        wid = lax.axis_index("s") * _NUM_CORES + lax.axis_index("c")
        pltpu.sync_copy(w_hbm, wv)
        nv = base_nv + jnp.where(wid < extra, 1, 0)
        base = wid * base_nv + jnp.minimum(wid, extra)

        def block(b, _):
            sv = base + jnp.minimum(b * _VB, nv - _VB)
            e0 = sv * _L
            pltpu.sync_copy(idx_hbm.at[pl.ds(e0, _VB * _L)], idxbuf)

            def vec(v, _):
                for u in range(_UNROLL):
                    off = (v * _UNROLL + u) * _L
                    idx = idxbuf[pl.ds(off, _L)]
                    outbuf[pl.ds(off, _L)] = plsc.load_gather(wv, [idx])
                return _

            lax.fori_loop(0, _VB // _UNROLL, vec, None, unroll=False)
            pltpu.sync_copy(outbuf, out_hbm.at[pl.ds(e0, _VB * _L)])
            return _

        lax.fori_loop(0, nb, block, None, unroll=False)

    return sc_gather


def kernel(weight, indices):
    total = indices.size
    wpad = jnp.pad(weight.astype(jnp.float32), (0, _L - weight.shape[0]))
    iflat = indices.reshape(total)
    out = _make_sc_gather(total)(wpad, iflat)
    return out.reshape(indices.shape)


# speedup vs baseline: 1.0714x; 1.0714x over previous
import functools

import jax
import jax.numpy as jnp
from jax import lax
from jax.experimental import pallas as pl
from jax.experimental.pallas import tpu as pltpu
from jax.experimental.pallas import tpu_sc as plsc

_NUM_CORES = 2
_NUM_SUBCORES = 16
_NW = _NUM_CORES * _NUM_SUBCORES
_L = 16

_VB = 128
_UNROLL = 8


def _make_sc_gather(total: int):
    assert total % _L == 0
    nv_total = total // _L
    base_nv = nv_total // _NW
    extra = nv_total % _NW
    max_nv = base_nv + (1 if extra else 0)
    nb = -(-max_nv // _VB)
    nb += nb % 2
    assert base_nv >= _VB and nb >= 4

    mesh = plsc.VectorSubcoreMesh(
        core_axis_name="c", subcore_axis_name="s", num_cores=_NUM_CORES
    )

    @functools.partial(
        pl.kernel,
        out_type=jax.ShapeDtypeStruct((total,), jnp.float32),
        mesh=mesh,
        scratch_types=[
            pltpu.VMEM((_L,), jnp.float32),
            [pltpu.VMEM((_VB * _L,), jnp.int32)] * 2,
            [pltpu.VMEM((_VB * _L,), jnp.float32)] * 2,
            [pltpu.SemaphoreType.DMA] * 2,
            [pltpu.SemaphoreType.DMA] * 2,
        ],
        compiler_params=pltpu.CompilerParams(needs_layout_passes=False),
    )
    def sc_gather(w_hbm, idx_hbm, out_hbm, wv, idxbuf, outbuf, isem, osem):
        wid = lax.axis_index("s") * _NUM_CORES + lax.axis_index("c")
        pltpu.sync_copy(w_hbm, wv)
        nv = base_nv + jnp.where(wid < extra, 1, 0)
        base = wid * base_nv + jnp.minimum(wid, extra)

        def e0(b):
            return (base + jnp.minimum(b * _VB, nv - _VB)) * _L

        def idx_at(b):
            return idx_hbm.at[pl.ds(e0(b), _VB * _L)]

        def out_at(b):
            return out_hbm.at[pl.ds(e0(b), _VB * _L)]

        def gather_block(p):
            @plsc.parallel_loop(0, _VB * _L, _UNROLL * _L, unroll=_UNROLL)
            def _(off):
                idx = idxbuf[p][pl.ds(off, _L)]
                outbuf[p][pl.ds(off, _L)] = plsc.load_gather(wv, [idx])

        pltpu.async_copy(idx_at(0), idxbuf[0], isem[0])
        pltpu.async_copy(idx_at(1), idxbuf[1], isem[1])

        for p in range(2):
            pltpu.make_async_copy(idx_at(p), idxbuf[p], isem[p]).wait()
            gather_block(p)
            pltpu.async_copy(outbuf[p], out_at(p), osem[p])
            pltpu.async_copy(idx_at(p + 2), idxbuf[p], isem[p])

        def bb_body(bb, _):
            b = bb * 2
            for p in range(2):
                pltpu.make_async_copy(outbuf[p], out_at(b + p - 2), osem[p]).wait()
                pltpu.make_async_copy(idx_at(b + p), idxbuf[p], isem[p]).wait()
                gather_block(p)
                pltpu.async_copy(outbuf[p], out_at(b + p), osem[p])
                pltpu.async_copy(idx_at(b + p + 2), idxbuf[p], isem[p])
            return _

        lax.fori_loop(1, nb // 2, bb_body, None, unroll=False)

        for p in range(2):
            pltpu.make_async_copy(idx_at(nb + p), idxbuf[p], isem[p]).wait()
            pltpu.make_async_copy(outbuf[p], out_at(nb + p - 2), osem[p]).wait()

    return sc_gather


def kernel(weight, indices):
    total = indices.size
    wpad = jnp.pad(weight.astype(jnp.float32), (0, _L - weight.shape[0]))
    iflat = indices.reshape(total)
    out = _make_sc_gather(total)(wpad, iflat)
    return out.reshape(indices.shape)


# speedup vs baseline: 13.6481x; 12.7391x over previous
import functools

import jax
import jax.numpy as jnp
from jax import lax
from jax.experimental import pallas as pl
from jax.experimental.pallas import tpu as pltpu
from jax.experimental.pallas import tpu_sc as plsc

_NUM_CORES = 2
_NUM_SUBCORES = 16
_NW = _NUM_CORES * _NUM_SUBCORES
_L = 16

_C = 2048
_UNROLL = 8


def _make_sc_gather(n_rows: int, n_cols: int):
    assert n_rows == 9
    chunk = -(-n_cols // _NW)
    chunk = -(-chunk // 128) * 128
    assert (n_cols - chunk) % 128 == 0
    nb = -(-chunk // _C)
    nb += nb % 2
    last = chunk - _C
    assert chunk <= n_cols and last % 128 == 0 and nb >= 4

    mesh = plsc.VectorSubcoreMesh(
        core_axis_name="c", subcore_axis_name="s", num_cores=_NUM_CORES
    )

    @functools.partial(
        pl.kernel,
        out_type=jax.ShapeDtypeStruct((n_rows, n_cols), jnp.float32),
        mesh=mesh,
        scratch_types=[
            pltpu.VMEM((_L,), jnp.float32),
            [pltpu.VMEM((8, _C), jnp.int32)] * 2,
            [pltpu.VMEM((1, _C), jnp.int32)] * 2,
            [pltpu.VMEM((8, _C), jnp.float32)] * 2,
            [pltpu.VMEM((1, _C), jnp.float32)] * 2,
            [pltpu.SemaphoreType.DMA] * 2,
            [pltpu.SemaphoreType.DMA] * 2,
        ],
        compiler_params=pltpu.CompilerParams(needs_layout_passes=False),
    )
    def sc_gather(w_hbm, idx_hbm, out_hbm, wv, ib8, ib1, ob8, ob1, isem, osem):
        wid = lax.axis_index("s") * _NUM_CORES + lax.axis_index("c")
        pltpu.sync_copy(w_hbm, wv)
        base = jnp.minimum(wid * chunk, n_cols - chunk)

        def c0(b):
            return base + jnp.minimum(b * _C, last)

        def in_at(b):
            c = c0(b)
            return (
                idx_hbm.at[pl.ds(0, 8), pl.ds(c, _C)],
                idx_hbm.at[pl.ds(8, 1), pl.ds(c, _C)],
            )

        def out_at(b):
            c = c0(b)
            return (
                out_hbm.at[pl.ds(0, 8), pl.ds(c, _C)],
                out_hbm.at[pl.ds(8, 1), pl.ds(c, _C)],
            )

        def start_in(b, p):
            s8, s1 = in_at(b)
            pltpu.async_copy(s8, ib8[p], isem[p])
            pltpu.async_copy(s1, ib1[p], isem[p])

        def wait_in(b, p):
            s8, s1 = in_at(b)
            pltpu.make_async_copy(s8, ib8[p], isem[p]).wait()
            pltpu.make_async_copy(s1, ib1[p], isem[p]).wait()

        def start_out(b, p):
            d8, d1 = out_at(b)
            pltpu.async_copy(ob8[p], d8, osem[p])
            pltpu.async_copy(ob1[p], d1, osem[p])

        def wait_out(b, p):
            d8, d1 = out_at(b)
            pltpu.make_async_copy(ob8[p], d8, osem[p]).wait()
            pltpu.make_async_copy(ob1[p], d1, osem[p]).wait()

        def gather_block(p):
            for r in range(8):
                @plsc.parallel_loop(0, _C, _L, unroll=_UNROLL)
                def _(o):
                    idx = ib8[p][r, pl.ds(o, _L)]
                    ob8[p][r, pl.ds(o, _L)] = plsc.load_gather(wv, [idx])

            @plsc.parallel_loop(0, _C, _L, unroll=_UNROLL)
            def _(o):
                idx = ib1[p][0, pl.ds(o, _L)]
                ob1[p][0, pl.ds(o, _L)] = plsc.load_gather(wv, [idx])

        start_in(0, 0)
        start_in(1, 1)

        for p in range(2):
            wait_in(p, p)
            gather_block(p)
            start_out(p, p)
            start_in(p + 2, p)

        def bb_body(bb, _):
            b = bb * 2
            for p in range(2):
                wait_out(b + p - 2, p)
                wait_in(b + p, p)
                gather_block(p)
                start_out(b + p, p)
                start_in(b + p + 2, p)
            return _

        lax.fori_loop(1, nb // 2, bb_body, None, unroll=False)

        for p in range(2):
            wait_in(nb + p, p)
            wait_out(nb + p - 2, p)

    return sc_gather


def kernel(weight, indices):
    wpad = jnp.pad(weight.astype(jnp.float32), (0, _L - weight.shape[0]))
    it = indices.T
    out_t = _make_sc_gather(*it.shape)(wpad, it)
    return out_t.T
